# baseline (speedup 1.0000x reference)
import jax
import jax.numpy as jnp
from jax import lax
from jax.experimental import pallas as pl
from jax.experimental.pallas import tpu as pltpu
from jax.experimental.pallas import tpu_sc as plsc

N = 10000
E = 320000
D = 128
NC, NS = 2, 16
NW = NC * NS
EPW = E // NW
C = 125
NCHUNK = EPW // C
RPT = N // NS
RC = RPT // C
NPAD = 10240
HPT = NPAD // NS

_mesh = plsc.VectorSubcoreMesh(core_axis_name="c", subcore_axis_name="s")
_sc_params = pltpu.CompilerParams(use_tc_tiling_on_sc=False)


def _hist_body(ei_hbm, out_hbm, dstv, ones_v, zrow_v, slab_v, deg_v, acc, sem):
    c = lax.axis_index("c")
    s = lax.axis_index("s")

    def fill(j, _):
        ones_v[j] = jnp.full((16,), 1.0, jnp.float32)
        return 0

    lax.fori_loop(0, C, fill, 0)

    def zfill(j, _):
        zrow_v[j] = jnp.zeros((16,), jnp.float32)
        return 0

    lax.fori_loop(0, 128, zfill, 0)

    pltpu.sync_copy(ei_hbm.at[1, c, s], dstv)

    def zslab(k, _):
        pltpu.sync_copy(zrow_v, acc.at[pl.ds(s * HPT + k * 128, 128)])
        return 0

    lax.fori_loop(0, HPT // 128, zslab, 0)
    plsc.subcore_barrier()

    G = 8

    def grp(g, _):
        for t in range(G):
            pltpu.async_copy(ones_v, acc.at[dstv.at[g * G + t]], sem, add=True)
        for t in range(G):
            pltpu.make_async_copy(ones_v, acc.at[pl.ds(0, C)], sem).wait()
        return 0

    lax.fori_loop(0, NCHUNK // G, grp, 0)
    plsc.subcore_barrier()

    pltpu.sync_copy(acc.at[pl.ds(s * HPT, HPT)], slab_v)
    lane = lax.iota(jnp.int32, 16)

    def ext(g, _):
        v = jnp.zeros((16,), jnp.float32)
        for r in range(16):
            v = jnp.where(lane == r, slab_v[g * 16 + r], v)
        deg_v[pl.ds(g * 16, 16)] = v
        return 0

    lax.fori_loop(0, HPT // 16, ext, 0)
    pltpu.sync_copy(deg_v, out_hbm.at[pl.ds(c * NPAD + s * HPT, HPT)])


_hist = pl.kernel(
    _hist_body,
    out_type=jax.ShapeDtypeStruct((NC * NPAD,), jnp.float32),
    mesh=_mesh,
    compiler_params=_sc_params,
    scratch_types=[
        pltpu.VMEM((NCHUNK, C), jnp.int32),
        pltpu.VMEM((C, 16), jnp.float32),
        pltpu.VMEM((128, 16), jnp.float32),
        pltpu.VMEM((HPT, 16), jnp.float32),
        pltpu.VMEM((HPT,), jnp.float32),
        pltpu.VMEM_SHARED((NPAD, 16), jnp.float32),
        pltpu.SemaphoreType.DMA,
    ],
)


NBUF = 5
NGRP = NCHUNK // NBUF


def _scatter_body(y_hbm, ei_hbm, out_hbm, srcv, dstv, r0b, r1b, r2b, r3b, r4b, acc,
                  s0, s1, s2, s3, s4):
    c = lax.axis_index("c")
    s = lax.axis_index("s")
    bufs = (r0b, r1b, r2b, r3b, r4b)
    sems = (s0, s1, s2, s3, s4)

    @pl.when(c == 0)
    def _():
        def yslab(k, _):
            r0 = s * RPT + k * C
            pltpu.sync_copy(y_hbm.at[pl.ds(r0, C)], r0b)
            pltpu.sync_copy(r0b, acc.at[pl.ds(r0, C)])
            return 0

        lax.fori_loop(0, RC, yslab, 0)

    @pl.when(c == 1)
    def _():
        def zfill(j, _):
            for k in range(D // 32):
                r0b[j, pl.ds(k * 32, 32)] = jnp.zeros((32,), jnp.bfloat16)
            return 0

        lax.fori_loop(0, C, zfill, 0)

        def zslab(k, _):
            pltpu.sync_copy(r0b, acc.at[pl.ds(s * RPT + k * C, C)])
            return 0

        lax.fori_loop(0, RC, zslab, 0)

    pltpu.sync_copy(ei_hbm.at[0, c, s], srcv)
    pltpu.sync_copy(ei_hbm.at[1, c, s], dstv)
    plsc.subcore_barrier()

    for t in range(NBUF):
        pltpu.async_copy(y_hbm.at[srcv.at[t]], bufs[t], sems[t])

    def grp(jj, _):
        j0 = jj * NBUF
        for t in range(NBUF):
            pltpu.make_async_copy(y_hbm.at[pl.ds(0, C)], bufs[t], sems[t]).wait()
            pltpu.sync_copy(bufs[t], acc.at[dstv.at[j0 + t]], add=True)

            @pl.when(jj < NGRP - 1)
            def _():
                pltpu.async_copy(y_hbm.at[srcv.at[j0 + t + NBUF]], bufs[t], sems[t])

        return 0

    lax.fori_loop(0, NGRP, grp, 0)
    plsc.subcore_barrier()

    def cpout(k, _):
        r0 = s * RPT + k * C
        pltpu.sync_copy(acc.at[pl.ds(r0, C)], out_hbm.at[c, pl.ds(r0, C)])
        return 0

    lax.fori_loop(0, RC, cpout, 0)


_scatter = pl.kernel(
    _scatter_body,
    out_type=jax.ShapeDtypeStruct((NC, N, D), jnp.bfloat16),
    mesh=_mesh,
    compiler_params=_sc_params,
    scratch_types=[
        pltpu.VMEM((NCHUNK, C), jnp.int32),
        pltpu.VMEM((NCHUNK, C), jnp.int32),
        pltpu.VMEM((C, D), jnp.bfloat16),
        pltpu.VMEM((C, D), jnp.bfloat16),
        pltpu.VMEM((C, D), jnp.bfloat16),
        pltpu.VMEM((C, D), jnp.bfloat16),
        pltpu.VMEM((C, D), jnp.bfloat16),
        pltpu.VMEM_SHARED((N, D), jnp.bfloat16),
        pltpu.SemaphoreType.DMA,
        pltpu.SemaphoreType.DMA,
        pltpu.SemaphoreType.DMA,
        pltpu.SemaphoreType.DMA,
        pltpu.SemaphoreType.DMA,
    ],
)


def _mm_body(x_ref, w_ref, xw_ref):
    xw_ref[...] = jnp.dot(x_ref[...], w_ref[...], preferred_element_type=jnp.float32)


_mm = pl.pallas_call(
    _mm_body,
    out_shape=jax.ShapeDtypeStruct((N, D), jnp.float32),
)


def _scale_body(xw_ref, h_ref, y_ref, dinv_ref):
    h = h_ref[...]
    deg = 1.0 + h[0:NPAD] + h[NPAD : 2 * NPAD]
    dinv = lax.rsqrt(deg)[0:N]
    y_ref[...] = (xw_ref[...] * dinv[:, None]).astype(jnp.bfloat16)
    dinv_ref[...] = dinv


_scale = pl.pallas_call(
    _scale_body,
    out_shape=(
        jax.ShapeDtypeStruct((N, D), jnp.bfloat16),
        jax.ShapeDtypeStruct((N,), jnp.float32),
    ),
)


def _bn_body(p_ref, dinv_ref, b_ref, g_ref, bt_ref, o_ref):
    tot = p_ref[0].astype(jnp.float32) + p_ref[1].astype(jnp.float32)
    pre = tot * dinv_ref[...][:, None] + b_ref[...][None, :]
    mean = jnp.mean(pre, axis=0)
    cen = pre - mean[None, :]
    var = jnp.mean(cen * cen, axis=0)
    o = cen * lax.rsqrt(var + 1e-5)[None, :] * g_ref[...][None, :] + bt_ref[...][None, :]
    o_ref[...] = jnp.where(o >= 0, o, 0.01 * o)


_bn = pl.pallas_call(
    _bn_body,
    out_shape=jax.ShapeDtypeStruct((N, D), jnp.float32),
)


def kernel(x, edge_index, W, b, gamma, beta):
    ei = jnp.reshape(edge_index.astype(jnp.int32), (2, NC, NS, NCHUNK, C))
    hist = _hist(ei)
    xw = _mm(x, W)
    y, dinv = _scale(xw, hist)
    p = _scatter(y, ei)
    return _bn(p, dinv, b, gamma, beta)


# speedup vs baseline: 1.0252x; 1.0252x over previous
import jax
import jax.numpy as jnp
from jax import lax
from jax.experimental import pallas as pl
from jax.experimental.pallas import tpu as pltpu
from jax.experimental.pallas import tpu_sc as plsc

N = 10000
E = 320000
D = 128
NC, NS = 2, 16
NW = NC * NS
CE = 128
NCH = E // CE
BASE = NCH // NW
EXTRA = NCH % NW
MAXCH = BASE + 1
NPAD = 10240
HPT = NPAD // NS

_mesh = plsc.VectorSubcoreMesh(core_axis_name="c", subcore_axis_name="s")
_sc_params = pltpu.CompilerParams(use_tc_tiling_on_sc=False)


def _hist_body(ei_hbm, out_hbm, dstv, ones_v, zrow_v, slab_v, deg_v, acc, sem):
    c = lax.axis_index("c")
    s = lax.axis_index("s")
    w = c * NS + s
    q0 = BASE * w + jnp.minimum(w, EXTRA)
    ncw = BASE + (w < EXTRA).astype(jnp.int32)

    def fill(j, _):
        ones_v[j] = jnp.full((16,), 1.0, jnp.float32)
        return 0

    lax.fori_loop(0, CE, fill, 0)

    def zfill(j, _):
        zrow_v[j] = jnp.zeros((16,), jnp.float32)
        return 0

    lax.fori_loop(0, 128, zfill, 0)

    pltpu.sync_copy(ei_hbm.at[pl.ds(NCH + q0, BASE)], dstv.at[pl.ds(0, BASE)])

    @pl.when(ncw > BASE)
    def _():
        pltpu.sync_copy(ei_hbm.at[pl.ds(NCH + q0 + BASE, 1)], dstv.at[pl.ds(BASE, 1)])

    def zslab(k, _):
        pltpu.sync_copy(zrow_v, acc.at[pl.ds(s * HPT + k * 128, 128)])
        return 0

    lax.fori_loop(0, HPT // 128, zslab, 0)
    plsc.subcore_barrier()

    G = 8

    def grp(g, _):
        for t in range(G):
            idx = g * G + t

            @pl.when(idx < ncw)
            def _():
                pltpu.async_copy(ones_v, acc.at[dstv.at[idx]], sem, add=True)

        for t in range(G):
            idx = g * G + t

            @pl.when(idx < ncw)
            def _():
                pltpu.make_async_copy(ones_v, acc.at[pl.ds(0, CE)], sem).wait()

        return 0

    lax.fori_loop(0, (MAXCH + G - 1) // G, grp, 0)
    plsc.subcore_barrier()

    pltpu.sync_copy(acc.at[pl.ds(s * HPT, HPT)], slab_v)
    lane = lax.iota(jnp.int32, 16)

    def ext(g, _):
        v = jnp.zeros((16,), jnp.float32)
        for r in range(16):
            v = jnp.where(lane == r, slab_v[g * 16 + r], v)
        deg_v[pl.ds(g * 16, 16)] = v
        return 0

    lax.fori_loop(0, HPT // 16, ext, 0)
    pltpu.sync_copy(deg_v, out_hbm.at[pl.ds(c * NPAD + s * HPT, HPT)])


_hist = pl.kernel(
    _hist_body,
    out_type=jax.ShapeDtypeStruct((NC * NPAD,), jnp.float32),
    mesh=_mesh,
    compiler_params=_sc_params,
    scratch_types=[
        pltpu.VMEM((MAXCH, CE), jnp.int32),
        pltpu.VMEM((CE, 16), jnp.float32),
        pltpu.VMEM((128, 16), jnp.float32),
        pltpu.VMEM((HPT, 16), jnp.float32),
        pltpu.VMEM((HPT,), jnp.float32),
        pltpu.VMEM_SHARED((NPAD, 16), jnp.float32),
        pltpu.SemaphoreType.DMA,
    ],
)


NBUF = 4


def _scatter_body(y_hbm, ei_hbm, out_hbm, srcv, dstv, r0b, r1b, r2b, r3b, acc,
                  s0, s1, s2, s3):
    c = lax.axis_index("c")
    s = lax.axis_index("s")
    w = c * NS + s
    q0 = BASE * w + jnp.minimum(w, EXTRA)
    ncw = BASE + (w < EXTRA).astype(jnp.int32)
    bufs = (r0b, r1b, r2b, r3b)
    sems = (s0, s1, s2, s3)

    @pl.when(c == 0)
    def _():
        def yslab(k, _):
            r0 = s * HPT + k * 128
            pltpu.sync_copy(y_hbm.at[pl.ds(r0, 128)], r0b)
            pltpu.sync_copy(r0b, acc.at[pl.ds(r0, 128)])
            return 0

        lax.fori_loop(0, HPT // 128, yslab, 0)

    @pl.when(c == 1)
    def _():
        def zfill(j, _):
            for k in range(D // 32):
                r0b[j, pl.ds(k * 32, 32)] = jnp.zeros((32,), jnp.bfloat16)
            return 0

        lax.fori_loop(0, CE, zfill, 0)

        def zslab(k, _):
            pltpu.sync_copy(r0b, acc.at[pl.ds(s * HPT + k * 128, 128)])
            return 0

        lax.fori_loop(0, HPT // 128, zslab, 0)

    pltpu.sync_copy(ei_hbm.at[pl.ds(q0, BASE)], srcv.at[pl.ds(0, BASE)])
    pltpu.sync_copy(ei_hbm.at[pl.ds(NCH + q0, BASE)], dstv.at[pl.ds(0, BASE)])

    @pl.when(ncw > BASE)
    def _():
        pltpu.sync_copy(ei_hbm.at[pl.ds(q0 + BASE, 1)], srcv.at[pl.ds(BASE, 1)])
        pltpu.sync_copy(ei_hbm.at[pl.ds(NCH + q0 + BASE, 1)], dstv.at[pl.ds(BASE, 1)])

    plsc.subcore_barrier()

    for t in range(NBUF):
        pltpu.async_copy(y_hbm.at[srcv.at[t]], bufs[t], sems[t])

    def grp(jj, _):
        j0 = jj * NBUF
        for t in range(NBUF):
            idx = j0 + t

            @pl.when(idx < ncw)
            def _():
                pltpu.make_async_copy(y_hbm.at[pl.ds(0, CE)], bufs[t], sems[t]).wait()
                pltpu.sync_copy(bufs[t], acc.at[dstv.at[idx]], add=True)

            @pl.when(idx + NBUF < ncw)
            def _():
                pltpu.async_copy(y_hbm.at[srcv.at[idx + NBUF]], bufs[t], sems[t])

        return 0

    lax.fori_loop(0, (MAXCH + NBUF - 1) // NBUF, grp, 0)
    plsc.subcore_barrier()

    def cpout(k, _):
        r0 = s * HPT + k * 128
        pltpu.sync_copy(acc.at[pl.ds(r0, 128)], out_hbm.at[c, pl.ds(r0, 128)])
        return 0

    lax.fori_loop(0, HPT // 128, cpout, 0)


_scatter = pl.kernel(
    _scatter_body,
    out_type=jax.ShapeDtypeStruct((NC, NPAD, D), jnp.bfloat16),
    mesh=_mesh,
    compiler_params=_sc_params,
    scratch_types=[
        pltpu.VMEM((MAXCH, CE), jnp.int32),
        pltpu.VMEM((MAXCH, CE), jnp.int32),
        pltpu.VMEM((CE, D), jnp.bfloat16),
        pltpu.VMEM((CE, D), jnp.bfloat16),
        pltpu.VMEM((CE, D), jnp.bfloat16),
        pltpu.VMEM((CE, D), jnp.bfloat16),
        pltpu.VMEM_SHARED((NPAD, D), jnp.bfloat16),
        pltpu.SemaphoreType.DMA,
        pltpu.SemaphoreType.DMA,
        pltpu.SemaphoreType.DMA,
        pltpu.SemaphoreType.DMA,
    ],
)


def _mm_body(x_ref, w_ref, xw_ref):
    xw_ref[...] = jnp.dot(x_ref[...], w_ref[...], preferred_element_type=jnp.float32)


_mm = pl.pallas_call(
    _mm_body,
    out_shape=jax.ShapeDtypeStruct((N, D), jnp.float32),
)


def _scale_body(xw_ref, h_ref, y_ref, dinv_ref):
    h = h_ref[...]
    deg = 1.0 + h[0:NPAD] + h[NPAD : 2 * NPAD]
    dinv = lax.rsqrt(deg)[0:N]
    y_ref[0:N, :] = (xw_ref[...] * dinv[:, None]).astype(jnp.bfloat16)
    y_ref[N:NPAD, :] = jnp.zeros((NPAD - N, D), jnp.bfloat16)
    dinv_ref[...] = dinv


_scale = pl.pallas_call(
    _scale_body,
    out_shape=(
        jax.ShapeDtypeStruct((NPAD, D), jnp.bfloat16),
        jax.ShapeDtypeStruct((N,), jnp.float32),
    ),
)


def _bn_body(p_ref, dinv_ref, b_ref, g_ref, bt_ref, o_ref):
    tot = p_ref[0, 0:N, :].astype(jnp.float32) + p_ref[1, 0:N, :].astype(jnp.float32)
    pre = tot * dinv_ref[...][:, None] + b_ref[...][None, :]
    mean = jnp.mean(pre, axis=0)
    cen = pre - mean[None, :]
    var = jnp.mean(cen * cen, axis=0)
    o = cen * lax.rsqrt(var + 1e-5)[None, :] * g_ref[...][None, :] + bt_ref[...][None, :]
    o_ref[...] = jnp.where(o >= 0, o, 0.01 * o)


_bn = pl.pallas_call(
    _bn_body,
    out_shape=jax.ShapeDtypeStruct((N, D), jnp.float32),
)


def kernel(x, edge_index, W, b, gamma, beta):
    ei = jnp.reshape(edge_index.astype(jnp.int32), (2 * NCH, CE))
    hist = _hist(ei)
    xw = _mm(x, W)
    y, dinv = _scale(xw, hist)
    p = _scatter(y, ei)
    return _bn(p, dinv, b, gamma, beta)
